# baseline (speedup 1.0000x reference)
import functools

import jax
import jax.numpy as jnp
from jax import lax
from jax.experimental import pallas as pl
from jax.experimental.pallas import tpu as pltpu
from jax.experimental.pallas import tpu_sc as plsc

_EMBED = 64
_HIST = 50
_LANES = 16
_LOG_AGE_MEAN = 3.0


def _sc_user_gather(batch, n_workers, vocab):
    b_per_w = batch // n_workers
    k_inflight = 4
    mesh = plsc.VectorSubcoreMesh(core_axis_name="c", subcore_axis_name="s")

    @functools.partial(
        pl.kernel,
        out_type=jax.ShapeDtypeStruct((batch, _EMBED), jnp.float32),
        mesh=mesh,
        compiler_params=pltpu.CompilerParams(needs_layout_passes=False),
        scratch_types=[
            pltpu.VMEM((b_per_w + _LANES,), jnp.int32),
            pltpu.VMEM((2, k_inflight, _EMBED, 128), jnp.float32),
            pltpu.VMEM((b_per_w, _EMBED), jnp.float32),
            pltpu.SemaphoreType.DMA,
            pltpu.SemaphoreType.DMA,
        ],
    )
    def k(user_table_t, user_id, out, idx_v, tiles_v, rows_v, sem, sem2):
        wid = lax.axis_index("s") * 2 + lax.axis_index("c")
        base = wid * b_per_w
        pltpu.sync_copy(user_id.at[pl.ds(base, b_per_w)],
                        idx_v.at[pl.ds(0, b_per_w)])
        eight = _EMBED // _LANES

        n_rounds = b_per_w // k_inflight
        iota = lax.iota(jnp.int32, _LANES)

        def fire(r, buf):
            vec = idx_v[pl.ds(r * k_inflight, _LANES)]
            for l in range(k_inflight):
                tile = pl.multiple_of((vec[l] >> 7) << 7, 128)
                pltpu.async_copy(
                    user_table_t.at[:, pl.ds(tile, 128)],
                    tiles_v.at[buf, l], sem)

        def extract(r, buf):
            vec = idx_v[pl.ds(r * k_inflight, _LANES)]
            for l in range(k_inflight):
                pltpu.make_async_copy(
                    user_table_t.at[:, pl.ds(0, 128)],
                    tiles_v.at[buf, l], sem).wait()
            for l in range(k_inflight):
                col = jnp.full((_LANES,), vec[l] & 127, jnp.int32)
                for c in range(eight):
                    e_vec = c * _LANES + iota
                    vals = plsc.load_gather(tiles_v.at[buf, l], [e_vec, col])
                    rows_v[r * k_inflight + l, pl.ds(c * _LANES, _LANES)] = vals

        fire(0, 0)

        def body(r, _):
            buf = r & 1
            @pl.when(r + 1 < n_rounds)
            def _():
                fire(r + 1, (r + 1) & 1)
            extract(r, buf)
            return 0

        lax.fori_loop(0, n_rounds, body, 0, unroll=2)
        pltpu.sync_copy(rows_v, out.at[pl.ds(base, b_per_w)])

    return k


def _tc_row_major(vocab):
    bn = 512

    def body(xt_ref, o_ref):
        xt = xt_ref[...]
        rows = lax.broadcasted_iota(jnp.int32, (_EMBED, 128), 0)
        cols = lax.broadcasted_iota(jnp.int32, (_EMBED, 128), 1)
        eye = (rows == cols).astype(jnp.float32)
        o_ref[...] = lax.dot_general(
            xt, eye, (((0,), (0,)), ((), ())),
            preferred_element_type=jnp.float32,
            precision=lax.Precision.HIGHEST)

    def apply(tab_t):
        return pl.pallas_call(
            body,
            grid=(pl.cdiv(vocab, bn),),
            in_specs=[pl.BlockSpec((_EMBED, bn), lambda i: (0, i))],
            out_specs=pl.BlockSpec((bn, 128), lambda i: (i, 0)),
            out_shape=jax.ShapeDtypeStruct((vocab, 128), jnp.float32),
        )(tab_t)

    return apply


def _sc_gather(batch, n_workers):
    b_per_w = batch // n_workers
    mesh = plsc.VectorSubcoreMesh(core_axis_name="c", subcore_axis_name="s")

    @functools.partial(
        pl.kernel,
        out_type=jax.ShapeDtypeStruct((6, batch, _EMBED), jnp.float32),
        mesh=mesh,
        compiler_params=pltpu.CompilerParams(use_tc_tiling_on_sc=False),
        scratch_types=[
            pltpu.VMEM((3, b_per_w), jnp.int32),
            pltpu.VMEM((3, _HIST, b_per_w), jnp.int32),
            pltpu.VMEM((3, b_per_w, _EMBED), jnp.float32),
            pltpu.VMEM((3, b_per_w, _EMBED), jnp.float32),
            pltpu.SemaphoreType.DMA,
            pltpu.SemaphoreType.DMA,
        ],
    )
    def k(seed_id, seed_artist_id, seed_album_id,
          hist_item_t, hist_artist_t, hist_album_t,
          item_table, artist_table, album_table,
          out, idx_v, hidx_v, rows_v, acc_v, sem_u, sem_h):
        wid = lax.axis_index("s") * 2 + lax.axis_index("c")
        base = wid * b_per_w

        pltpu.sync_copy(seed_id.at[pl.ds(base, b_per_w)], idx_v.at[0])
        pltpu.sync_copy(seed_artist_id.at[pl.ds(base, b_per_w)], idx_v.at[1])
        pltpu.sync_copy(seed_album_id.at[pl.ds(base, b_per_w)], idx_v.at[2])
        pltpu.sync_copy(hist_item_t.at[:, pl.ds(base, b_per_w)], hidx_v.at[0])
        pltpu.sync_copy(hist_artist_t.at[:, pl.ds(base, b_per_w)], hidx_v.at[1])
        pltpu.sync_copy(hist_album_t.at[:, pl.ds(base, b_per_w)], hidx_v.at[2])

        zero = jnp.zeros((_LANES,), jnp.float32)

        def zero_body(i, _):
            for t in range(3):
                for c in range(_EMBED // _LANES):
                    acc_v[t, i, pl.ds(c * _LANES, _LANES)] = zero
            return 0

        lax.fori_loop(0, b_per_w, zero_body, 0)

        tables = (item_table, artist_table, album_table)

        ds = [pltpu.async_copy(tables[t].at[idx_v.at[t]], rows_v.at[t], sem_u)
              for t in range(3)]

        def issue_body(j, _):
            for t in range(3):
                pltpu.async_copy(
                    tables[t].at[hidx_v.at[t, j]], acc_v.at[t], sem_h, add=True)
            return 0

        lax.fori_loop(0, _HIST, issue_body, 0)

        for d in ds:
            d.wait()
        for t in range(3):
            pltpu.sync_copy(rows_v.at[t], out.at[3 + t, pl.ds(base, b_per_w)])

        def drain_body(j, _):
            for t in range(3):
                pltpu.make_async_copy(
                    tables[t].at[hidx_v.at[t, j]], acc_v.at[t], sem_h).wait()
            return 0

        lax.fori_loop(0, _HIST, drain_body, 0)

        scale = jnp.full((_LANES,), 1.0 / _HIST, jnp.float32)

        def scale_body(i, _):
            for t in range(3):
                for c in range(_EMBED // _LANES):
                    sl = pl.ds(c * _LANES, _LANES)
                    acc_v[t, i, sl] = acc_v[t, i, sl] * scale
            return 0

        lax.fori_loop(0, b_per_w, scale_body, 0)
        for t in range(3):
            pltpu.sync_copy(acc_v.at[t], out.at[t, pl.ds(base, b_per_w)])

    return k


def _mlp_body(u_ref, x_ref, htl_ref, sl_ref, age_ref, w1_ref, ws_ref, b1_ref,
              w2_ref, b2_ref, out_ref):
    xcat = jnp.concatenate(
        [u_ref[...]] + [x_ref[t] for t in range(6)], axis=1)
    s = jnp.dot(xcat, w1_ref[...], preferred_element_type=jnp.float32,
                precision=lax.Precision.HIGHEST)
    hl = jnp.sum(htl_ref[...], axis=1, keepdims=True) * (1.0 / _HIST)
    age_n = jnp.log1p(age_ref[...]) - _LOG_AGE_MEAN
    s = s + hl * ws_ref[0:1, :] + sl_ref[...] * ws_ref[1:2, :] \
        + age_n * ws_ref[2:3, :]
    h = jnp.maximum(s + b1_ref[...], 0.0)
    out_ref[...] = jnp.dot(h, w2_ref[...], preferred_element_type=jnp.float32,
                           precision=lax.Precision.HIGHEST) + b2_ref[...]


def kernel(user_id, history_id, history_artist_id, history_album_id,
           history_track_length, seed_id, seed_artist_id, seed_album_id,
           seed_track_length, age,
           user_table, item_table, artist_table, album_table,
           W1, b1, W2, b2):
    batch = user_id.shape[0]
    n_workers = 32
    user_rows = _sc_user_gather(batch, n_workers, user_table.shape[0])(
        user_table.T, user_id)

    def _row_major(tab):
        vocab = tab.shape[0]
        rm = _tc_row_major(vocab)(tab.T)
        return jnp.reshape(rm, (2 * vocab, _EMBED))

    item_rm = _row_major(item_table)
    artist_rm = _row_major(artist_table)
    album_rm = _row_major(album_table)
    emb = _sc_gather(batch, n_workers)(
        seed_id * 2, seed_artist_id * 2, seed_album_id * 2,
        history_id.T * 2, history_artist_id.T * 2, history_album_id.T * 2,
        item_rm, artist_rm, album_rm)

    w1_emb = jnp.concatenate([W1[0:256], W1[257:449]], axis=0)
    w_scal = jnp.stack([W1[256], W1[449], W1[450]], axis=0)

    hidden1 = W1.shape[1]
    hidden2 = W2.shape[1]
    bm = 512
    grid = (batch // bm,)
    out = pl.pallas_call(
        _mlp_body,
        grid=grid,
        in_specs=[
            pl.BlockSpec((bm, _EMBED), lambda i: (i, 0)),
            pl.BlockSpec((6, bm, _EMBED), lambda i: (0, i, 0)),
            pl.BlockSpec((bm, _HIST), lambda i: (i, 0)),
            pl.BlockSpec((bm, 1), lambda i: (i, 0)),
            pl.BlockSpec((bm, 1), lambda i: (i, 0)),
            pl.BlockSpec((7 * _EMBED, hidden1), lambda i: (0, 0)),
            pl.BlockSpec((3, hidden1), lambda i: (0, 0)),
            pl.BlockSpec((1, hidden1), lambda i: (0, 0)),
            pl.BlockSpec((hidden1, hidden2), lambda i: (0, 0)),
            pl.BlockSpec((1, hidden2), lambda i: (0, 0)),
        ],
        out_specs=pl.BlockSpec((bm, hidden2), lambda i: (i, 0)),
        out_shape=jax.ShapeDtypeStruct((batch, hidden2), jnp.float32),
    )(user_rows, emb, history_track_length, seed_track_length[:, None],
      age[:, None], w1_emb, w_scal, b1[None, :], W2, b2[None, :])
    return out


# speedup vs baseline: 2.5047x; 2.5047x over previous
import functools

import jax
import jax.numpy as jnp
from jax import lax
from jax.experimental import pallas as pl
from jax.experimental.pallas import tpu as pltpu
from jax.experimental.pallas import tpu_sc as plsc

_EMBED = 64
_HIST = 50
_LANES = 16
_LOG_AGE_MEAN = 3.0


def _sc_user_gather(batch, n_workers, vocab):
    b_per_w = batch // n_workers
    k_inflight = 4
    mesh = plsc.VectorSubcoreMesh(core_axis_name="c", subcore_axis_name="s")

    @functools.partial(
        pl.kernel,
        out_type=jax.ShapeDtypeStruct((batch, _EMBED), jnp.float32),
        mesh=mesh,
        compiler_params=pltpu.CompilerParams(needs_layout_passes=False),
        scratch_types=[
            pltpu.VMEM((b_per_w + _LANES,), jnp.int32),
            pltpu.VMEM((2, k_inflight, _EMBED, 128), jnp.float32),
            pltpu.VMEM((b_per_w, _EMBED), jnp.float32),
            pltpu.SemaphoreType.DMA,
            pltpu.SemaphoreType.DMA,
        ],
    )
    def k(user_table_t, user_id, out, idx_v, tiles_v, rows_v, sem, sem2):
        wid = lax.axis_index("s") * 2 + lax.axis_index("c")
        base = wid * b_per_w
        pltpu.sync_copy(user_id.at[pl.ds(base, b_per_w)],
                        idx_v.at[pl.ds(0, b_per_w)])
        eight = _EMBED // _LANES

        n_rounds = b_per_w // k_inflight
        iota = lax.iota(jnp.int32, _LANES)

        def fire(r, buf):
            vec = idx_v[pl.ds(r * k_inflight, _LANES)]
            for l in range(k_inflight):
                tile = pl.multiple_of((vec[l] >> 7) << 7, 128)
                pltpu.async_copy(
                    user_table_t.at[:, pl.ds(tile, 128)],
                    tiles_v.at[buf, l], sem)

        def extract(r, buf):
            vec = idx_v[pl.ds(r * k_inflight, _LANES)]
            for l in range(k_inflight):
                pltpu.make_async_copy(
                    user_table_t.at[:, pl.ds(0, 128)],
                    tiles_v.at[buf, l], sem).wait()
            for l in range(k_inflight):
                col = jnp.full((_LANES,), vec[l] & 127, jnp.int32)
                for c in range(eight):
                    e_vec = c * _LANES + iota
                    vals = plsc.load_gather(tiles_v.at[buf, l], [e_vec, col])
                    rows_v[r * k_inflight + l, pl.ds(c * _LANES, _LANES)] = vals

        fire(0, 0)

        def body(r, _):
            buf = r & 1
            @pl.when(r + 1 < n_rounds)
            def _():
                fire(r + 1, (r + 1) & 1)
            extract(r, buf)
            return 0

        lax.fori_loop(0, n_rounds, body, 0, unroll=2)
        pltpu.sync_copy(rows_v, out.at[pl.ds(base, b_per_w)])

    return k


def _tc_row_major(vocab):
    bn = 4096

    def body(xt_ref, o_ref):
        xt = xt_ref[...]
        rows = lax.broadcasted_iota(jnp.int32, (_EMBED, 128), 0)
        cols = lax.broadcasted_iota(jnp.int32, (_EMBED, 128), 1)
        eye = (rows == cols).astype(jnp.float32)
        o_ref[...] = lax.dot_general(
            xt, eye, (((0,), (0,)), ((), ())),
            preferred_element_type=jnp.float32,
            precision=lax.Precision.HIGHEST)

    def apply(tab_t):
        return pl.pallas_call(
            body,
            grid=(pl.cdiv(vocab, bn),),
            in_specs=[pl.BlockSpec((_EMBED, bn), lambda i: (0, i))],
            out_specs=pl.BlockSpec((bn, 128), lambda i: (i, 0)),
            out_shape=jax.ShapeDtypeStruct((vocab, 128), jnp.float32),
        )(tab_t)

    return apply


def _sc_gather(batch, n_workers):
    b_per_w = batch // n_workers
    mesh = plsc.VectorSubcoreMesh(core_axis_name="c", subcore_axis_name="s")

    @functools.partial(
        pl.kernel,
        out_type=jax.ShapeDtypeStruct((6, batch, _EMBED), jnp.float32),
        mesh=mesh,
        compiler_params=pltpu.CompilerParams(use_tc_tiling_on_sc=False),
        scratch_types=[
            pltpu.VMEM((3, b_per_w), jnp.int32),
            pltpu.VMEM((3, _HIST, b_per_w), jnp.int32),
            pltpu.VMEM((3, b_per_w, _EMBED), jnp.float32),
            pltpu.VMEM((3, b_per_w, _EMBED), jnp.float32),
            pltpu.SemaphoreType.DMA,
            pltpu.SemaphoreType.DMA,
        ],
    )
    def k(seed_id, seed_artist_id, seed_album_id,
          hist_item_t, hist_artist_t, hist_album_t,
          item_table, artist_table, album_table,
          out, idx_v, hidx_v, rows_v, acc_v, sem_u, sem_h):
        wid = lax.axis_index("s") * 2 + lax.axis_index("c")
        base = wid * b_per_w

        pltpu.sync_copy(seed_id.at[pl.ds(base, b_per_w)], idx_v.at[0])
        pltpu.sync_copy(seed_artist_id.at[pl.ds(base, b_per_w)], idx_v.at[1])
        pltpu.sync_copy(seed_album_id.at[pl.ds(base, b_per_w)], idx_v.at[2])
        pltpu.sync_copy(hist_item_t.at[:, pl.ds(base, b_per_w)], hidx_v.at[0])
        pltpu.sync_copy(hist_artist_t.at[:, pl.ds(base, b_per_w)], hidx_v.at[1])
        pltpu.sync_copy(hist_album_t.at[:, pl.ds(base, b_per_w)], hidx_v.at[2])

        zero = jnp.zeros((_LANES,), jnp.float32)

        def zero_body(i, _):
            for t in range(3):
                for c in range(_EMBED // _LANES):
                    acc_v[t, i, pl.ds(c * _LANES, _LANES)] = zero
            return 0

        lax.fori_loop(0, b_per_w, zero_body, 0)

        tables = (item_table, artist_table, album_table)

        ds = [pltpu.async_copy(tables[t].at[idx_v.at[t]], rows_v.at[t], sem_u)
              for t in range(3)]

        def issue_body(j, _):
            for t in range(3):
                pltpu.async_copy(
                    tables[t].at[hidx_v.at[t, j]], acc_v.at[t], sem_h, add=True)
            return 0

        lax.fori_loop(0, _HIST, issue_body, 0)

        for d in ds:
            d.wait()
        for t in range(3):
            pltpu.sync_copy(rows_v.at[t], out.at[3 + t, pl.ds(base, b_per_w)])

        def drain_body(j, _):
            for t in range(3):
                pltpu.make_async_copy(
                    tables[t].at[hidx_v.at[t, j]], acc_v.at[t], sem_h).wait()
            return 0

        lax.fori_loop(0, _HIST, drain_body, 0)

        scale = jnp.full((_LANES,), 1.0 / _HIST, jnp.float32)

        def scale_body(i, _):
            for t in range(3):
                for c in range(_EMBED // _LANES):
                    sl = pl.ds(c * _LANES, _LANES)
                    acc_v[t, i, sl] = acc_v[t, i, sl] * scale
            return 0

        lax.fori_loop(0, b_per_w, scale_body, 0)
        for t in range(3):
            pltpu.sync_copy(acc_v.at[t], out.at[t, pl.ds(base, b_per_w)])

    return k


def _mlp_body(u_ref, x_ref, htl_ref, sl_ref, age_ref, w1_ref, ws_ref, b1_ref,
              w2_ref, b2_ref, out_ref):
    xcat = jnp.concatenate(
        [u_ref[...]] + [x_ref[t] for t in range(6)], axis=1)
    s = jnp.dot(xcat, w1_ref[...], preferred_element_type=jnp.float32,
                precision=lax.Precision.HIGHEST)
    hl = jnp.sum(htl_ref[...], axis=1, keepdims=True) * (1.0 / _HIST)
    age_n = jnp.log1p(age_ref[...]) - _LOG_AGE_MEAN
    s = s + hl * ws_ref[0:1, :] + sl_ref[...] * ws_ref[1:2, :] \
        + age_n * ws_ref[2:3, :]
    h = jnp.maximum(s + b1_ref[...], 0.0)
    out_ref[...] = jnp.dot(h, w2_ref[...], preferred_element_type=jnp.float32,
                           precision=lax.Precision.HIGHEST) + b2_ref[...]


def kernel(user_id, history_id, history_artist_id, history_album_id,
           history_track_length, seed_id, seed_artist_id, seed_album_id,
           seed_track_length, age,
           user_table, item_table, artist_table, album_table,
           W1, b1, W2, b2):
    batch = user_id.shape[0]
    n_workers = 32
    user_rows = _sc_user_gather(batch, n_workers, user_table.shape[0])(
        user_table.T, user_id)

    def _row_major(tab):
        vocab = tab.shape[0]
        rm = _tc_row_major(vocab)(tab.T)
        return jnp.reshape(rm, (2 * vocab, _EMBED))

    item_rm = _row_major(item_table)
    artist_rm = _row_major(artist_table)
    album_rm = _row_major(album_table)
    emb = _sc_gather(batch, n_workers)(
        seed_id * 2, seed_artist_id * 2, seed_album_id * 2,
        history_id.T * 2, history_artist_id.T * 2, history_album_id.T * 2,
        item_rm, artist_rm, album_rm)

    w1_emb = jnp.concatenate([W1[0:256], W1[257:449]], axis=0)
    w_scal = jnp.stack([W1[256], W1[449], W1[450]], axis=0)

    hidden1 = W1.shape[1]
    hidden2 = W2.shape[1]
    bm = 512
    grid = (batch // bm,)
    out = pl.pallas_call(
        _mlp_body,
        grid=grid,
        in_specs=[
            pl.BlockSpec((bm, _EMBED), lambda i: (i, 0)),
            pl.BlockSpec((6, bm, _EMBED), lambda i: (0, i, 0)),
            pl.BlockSpec((bm, _HIST), lambda i: (i, 0)),
            pl.BlockSpec((bm, 1), lambda i: (i, 0)),
            pl.BlockSpec((bm, 1), lambda i: (i, 0)),
            pl.BlockSpec((7 * _EMBED, hidden1), lambda i: (0, 0)),
            pl.BlockSpec((3, hidden1), lambda i: (0, 0)),
            pl.BlockSpec((1, hidden1), lambda i: (0, 0)),
            pl.BlockSpec((hidden1, hidden2), lambda i: (0, 0)),
            pl.BlockSpec((1, hidden2), lambda i: (0, 0)),
        ],
        out_specs=pl.BlockSpec((bm, hidden2), lambda i: (i, 0)),
        out_shape=jax.ShapeDtypeStruct((batch, hidden2), jnp.float32),
    )(user_rows, emb, history_track_length, seed_track_length[:, None],
      age[:, None], w1_emb, w_scal, b1[None, :], W2, b2[None, :])
    return out


# speedup vs baseline: 2.6767x; 1.0686x over previous
import functools

import jax
import jax.numpy as jnp
from jax import lax
from jax.experimental import pallas as pl
from jax.experimental.pallas import tpu as pltpu
from jax.experimental.pallas import tpu_sc as plsc

_EMBED = 64
_HIST = 50
_LANES = 16
_LOG_AGE_MEAN = 3.0


def _sc_user_gather(batch, n_workers, vocab):
    b_per_w = batch // n_workers
    k_inflight = 4
    mesh = plsc.VectorSubcoreMesh(core_axis_name="c", subcore_axis_name="s")

    @functools.partial(
        pl.kernel,
        out_type=jax.ShapeDtypeStruct((batch, _EMBED), jnp.float32),
        mesh=mesh,
        compiler_params=pltpu.CompilerParams(needs_layout_passes=False),
        scratch_types=[
            pltpu.VMEM((b_per_w + _LANES,), jnp.int32),
            pltpu.VMEM((2, k_inflight, _EMBED, 128), jnp.float32),
            pltpu.VMEM((b_per_w, _EMBED), jnp.float32),
            pltpu.SemaphoreType.DMA,
            pltpu.SemaphoreType.DMA,
        ],
    )
    def k(user_table_t, user_id, out, idx_v, tiles_v, rows_v, sem, sem2):
        wid = lax.axis_index("s") * 2 + lax.axis_index("c")
        base = wid * b_per_w
        pltpu.sync_copy(user_id.at[pl.ds(base, b_per_w)],
                        idx_v.at[pl.ds(0, b_per_w)])
        eight = _EMBED // _LANES

        n_rounds = b_per_w // k_inflight
        iota = lax.iota(jnp.int32, _LANES)

        def fire(r, buf):
            vec = idx_v[pl.ds(r * k_inflight, _LANES)]
            for l in range(k_inflight):
                tile = pl.multiple_of((vec[l] >> 7) << 7, 128)
                pltpu.async_copy(
                    user_table_t.at[:, pl.ds(tile, 128)],
                    tiles_v.at[buf, l], sem)

        def extract(r, buf):
            vec = idx_v[pl.ds(r * k_inflight, _LANES)]
            for l in range(k_inflight):
                pltpu.make_async_copy(
                    user_table_t.at[:, pl.ds(0, 128)],
                    tiles_v.at[buf, l], sem).wait()
            for l in range(k_inflight):
                col = jnp.full((_LANES,), vec[l] & 127, jnp.int32)
                for c in range(eight):
                    e_vec = c * _LANES + iota
                    vals = plsc.load_gather(tiles_v.at[buf, l], [e_vec, col])
                    rows_v[r * k_inflight + l, pl.ds(c * _LANES, _LANES)] = vals

        fire(0, 0)

        def body(r, _):
            buf = r & 1
            @pl.when(r + 1 < n_rounds)
            def _():
                fire(r + 1, (r + 1) & 1)
            extract(r, buf)
            return 0

        lax.fori_loop(0, n_rounds, body, 0, unroll=2)
        pltpu.sync_copy(rows_v, out.at[pl.ds(base, b_per_w)])

    return k


def _tc_row_major(vocab):
    bn = 8192

    def body(xt_ref, o_ref):
        xt = xt_ref[...]
        rows = lax.broadcasted_iota(jnp.int32, (_EMBED, 128), 0)
        cols = lax.broadcasted_iota(jnp.int32, (_EMBED, 128), 1)
        eye = (rows == cols).astype(jnp.float32)
        o_ref[...] = lax.dot_general(
            xt, eye, (((0,), (0,)), ((), ())),
            preferred_element_type=jnp.float32,
            precision=lax.Precision.HIGHEST)

    def apply(tab_t):
        return pl.pallas_call(
            body,
            grid=(pl.cdiv(vocab, bn),),
            in_specs=[pl.BlockSpec((_EMBED, bn), lambda i: (0, i))],
            out_specs=pl.BlockSpec((bn, 128), lambda i: (i, 0)),
            out_shape=jax.ShapeDtypeStruct((vocab, 128), jnp.float32),
        )(tab_t)

    return apply


def _sc_gather(batch, n_workers):
    b_per_w = batch // n_workers
    mesh = plsc.VectorSubcoreMesh(core_axis_name="c", subcore_axis_name="s")

    @functools.partial(
        pl.kernel,
        out_type=jax.ShapeDtypeStruct((6, batch, _EMBED), jnp.float32),
        mesh=mesh,
        compiler_params=pltpu.CompilerParams(use_tc_tiling_on_sc=False),
        scratch_types=[
            pltpu.VMEM((3, b_per_w), jnp.int32),
            pltpu.VMEM((3, _HIST, b_per_w), jnp.int32),
            pltpu.VMEM((3, b_per_w, _EMBED), jnp.float32),
            pltpu.VMEM((3, b_per_w, _EMBED), jnp.float32),
            pltpu.SemaphoreType.DMA,
            pltpu.SemaphoreType.DMA,
        ],
    )
    def k(seed_id, seed_artist_id, seed_album_id,
          hist_item_t, hist_artist_t, hist_album_t,
          item_table, artist_table, album_table,
          out, idx_v, hidx_v, rows_v, acc_v, sem_u, sem_h):
        wid = lax.axis_index("s") * 2 + lax.axis_index("c")
        base = wid * b_per_w

        pltpu.sync_copy(seed_id.at[pl.ds(base, b_per_w)], idx_v.at[0])
        pltpu.sync_copy(seed_artist_id.at[pl.ds(base, b_per_w)], idx_v.at[1])
        pltpu.sync_copy(seed_album_id.at[pl.ds(base, b_per_w)], idx_v.at[2])
        pltpu.sync_copy(hist_item_t.at[:, pl.ds(base, b_per_w)], hidx_v.at[0])
        pltpu.sync_copy(hist_artist_t.at[:, pl.ds(base, b_per_w)], hidx_v.at[1])
        pltpu.sync_copy(hist_album_t.at[:, pl.ds(base, b_per_w)], hidx_v.at[2])

        zero = jnp.zeros((_LANES,), jnp.float32)

        def zero_body(i, _):
            for t in range(3):
                for c in range(_EMBED // _LANES):
                    acc_v[t, i, pl.ds(c * _LANES, _LANES)] = zero
            return 0

        lax.fori_loop(0, b_per_w, zero_body, 0)

        tables = (item_table, artist_table, album_table)

        ds = [pltpu.async_copy(tables[t].at[idx_v.at[t]], rows_v.at[t], sem_u)
              for t in range(3)]

        def issue_body(j, _):
            for t in range(3):
                pltpu.async_copy(
                    tables[t].at[hidx_v.at[t, j]], acc_v.at[t], sem_h, add=True)
            return 0

        lax.fori_loop(0, _HIST, issue_body, 0)

        for d in ds:
            d.wait()
        for t in range(3):
            pltpu.sync_copy(rows_v.at[t], out.at[3 + t, pl.ds(base, b_per_w)])

        def drain_body(j, _):
            for t in range(3):
                pltpu.make_async_copy(
                    tables[t].at[hidx_v.at[t, j]], acc_v.at[t], sem_h).wait()
            return 0

        lax.fori_loop(0, _HIST, drain_body, 0)

        scale = jnp.full((_LANES,), 1.0 / _HIST, jnp.float32)

        def scale_body(i, _):
            for t in range(3):
                for c in range(_EMBED // _LANES):
                    sl = pl.ds(c * _LANES, _LANES)
                    acc_v[t, i, sl] = acc_v[t, i, sl] * scale
            return 0

        lax.fori_loop(0, b_per_w, scale_body, 0)
        for t in range(3):
            pltpu.sync_copy(acc_v.at[t], out.at[t, pl.ds(base, b_per_w)])

    return k


def _mlp_body(u_ref, x_ref, htl_ref, sl_ref, age_ref, w1_ref, ws_ref, b1_ref,
              w2_ref, b2_ref, out_ref):
    xcat = jnp.concatenate(
        [u_ref[...]] + [x_ref[t] for t in range(6)], axis=1)
    s = jnp.dot(xcat, w1_ref[...], preferred_element_type=jnp.float32,
                precision=lax.Precision.HIGHEST)
    hl = jnp.sum(htl_ref[...], axis=1, keepdims=True) * (1.0 / _HIST)
    age_n = jnp.log1p(age_ref[...]) - _LOG_AGE_MEAN
    s = s + hl * ws_ref[0:1, :] + sl_ref[...] * ws_ref[1:2, :] \
        + age_n * ws_ref[2:3, :]
    h = jnp.maximum(s + b1_ref[...], 0.0)
    out_ref[...] = jnp.dot(h, w2_ref[...], preferred_element_type=jnp.float32,
                           precision=lax.Precision.HIGHEST) + b2_ref[...]


def kernel(user_id, history_id, history_artist_id, history_album_id,
           history_track_length, seed_id, seed_artist_id, seed_album_id,
           seed_track_length, age,
           user_table, item_table, artist_table, album_table,
           W1, b1, W2, b2):
    batch = user_id.shape[0]
    n_workers = 32
    user_rows = _sc_user_gather(batch, n_workers, user_table.shape[0])(
        user_table.T, user_id)

    def _row_major(tab):
        vocab = tab.shape[0]
        rm = _tc_row_major(vocab)(tab.T)
        return jnp.reshape(rm, (2 * vocab, _EMBED))

    item_rm = _row_major(item_table)
    artist_rm = _row_major(artist_table)
    album_rm = _row_major(album_table)
    emb = _sc_gather(batch, n_workers)(
        seed_id * 2, seed_artist_id * 2, seed_album_id * 2,
        history_id.T * 2, history_artist_id.T * 2, history_album_id.T * 2,
        item_rm, artist_rm, album_rm)

    w1_emb = jnp.concatenate([W1[0:256], W1[257:449]], axis=0)
    w_scal = jnp.stack([W1[256], W1[449], W1[450]], axis=0)

    hidden1 = W1.shape[1]
    hidden2 = W2.shape[1]
    bm = 512
    grid = (batch // bm,)
    out = pl.pallas_call(
        _mlp_body,
        grid=grid,
        in_specs=[
            pl.BlockSpec((bm, _EMBED), lambda i: (i, 0)),
            pl.BlockSpec((6, bm, _EMBED), lambda i: (0, i, 0)),
            pl.BlockSpec((bm, _HIST), lambda i: (i, 0)),
            pl.BlockSpec((bm, 1), lambda i: (i, 0)),
            pl.BlockSpec((bm, 1), lambda i: (i, 0)),
            pl.BlockSpec((7 * _EMBED, hidden1), lambda i: (0, 0)),
            pl.BlockSpec((3, hidden1), lambda i: (0, 0)),
            pl.BlockSpec((1, hidden1), lambda i: (0, 0)),
            pl.BlockSpec((hidden1, hidden2), lambda i: (0, 0)),
            pl.BlockSpec((1, hidden2), lambda i: (0, 0)),
        ],
        out_specs=pl.BlockSpec((bm, hidden2), lambda i: (i, 0)),
        out_shape=jax.ShapeDtypeStruct((batch, hidden2), jnp.float32),
    )(user_rows, emb, history_track_length, seed_track_length[:, None],
      age[:, None], w1_emb, w_scal, b1[None, :], W2, b2[None, :])
    return out


# speedup vs baseline: 3.7926x; 1.4169x over previous
import functools

import jax
import jax.numpy as jnp
from jax import lax
from jax.experimental import pallas as pl
from jax.experimental.pallas import tpu as pltpu
from jax.experimental.pallas import tpu_sc as plsc

_EMBED = 64
_HIST = 50
_LANES = 16
_LOG_AGE_MEAN = 3.0


def _sc_user_gather(batch, n_workers, vocab):
    b_per_w = batch // n_workers
    k_inflight = 4
    mesh = plsc.VectorSubcoreMesh(core_axis_name="c", subcore_axis_name="s")

    @functools.partial(
        pl.kernel,
        out_type=jax.ShapeDtypeStruct((batch, _EMBED), jnp.float32),
        mesh=mesh,
        compiler_params=pltpu.CompilerParams(needs_layout_passes=False),
        scratch_types=[
            pltpu.VMEM((b_per_w + _LANES,), jnp.int32),
            pltpu.VMEM((2, k_inflight, _EMBED, 128), jnp.float32),
            pltpu.VMEM((b_per_w, _EMBED), jnp.float32),
            pltpu.SemaphoreType.DMA,
            pltpu.SemaphoreType.DMA,
        ],
    )
    def k(user_table_t, user_id, out, idx_v, tiles_v, rows_v, sem, sem2):
        wid = lax.axis_index("s") * 2 + lax.axis_index("c")
        base = wid * b_per_w
        pltpu.sync_copy(user_id.at[pl.ds(base, b_per_w)],
                        idx_v.at[pl.ds(0, b_per_w)])
        eight = _EMBED // _LANES

        n_rounds = b_per_w // k_inflight
        iota = lax.iota(jnp.int32, _LANES)

        def fire(r, buf):
            vec = idx_v[pl.ds(r * k_inflight, _LANES)]
            for l in range(k_inflight):
                tile = pl.multiple_of((vec[l] >> 7) << 7, 128)
                pltpu.async_copy(
                    user_table_t.at[:, pl.ds(tile, 128)],
                    tiles_v.at[buf, l], sem)

        def extract(r, buf):
            vec = idx_v[pl.ds(r * k_inflight, _LANES)]
            for l in range(k_inflight):
                pltpu.make_async_copy(
                    user_table_t.at[:, pl.ds(0, 128)],
                    tiles_v.at[buf, l], sem).wait()
            for l in range(k_inflight):
                col = jnp.full((_LANES,), vec[l] & 127, jnp.int32)
                for c in range(eight):
                    e_vec = c * _LANES + iota
                    vals = plsc.load_gather(tiles_v.at[buf, l], [e_vec, col])
                    rows_v[r * k_inflight + l, pl.ds(c * _LANES, _LANES)] = vals

        fire(0, 0)

        def body(r, _):
            buf = r & 1
            @pl.when(r + 1 < n_rounds)
            def _():
                fire(r + 1, (r + 1) & 1)
            extract(r, buf)
            return 0

        lax.fori_loop(0, n_rounds, body, 0, unroll=2)
        pltpu.sync_copy(rows_v, out.at[pl.ds(base, b_per_w)])

    return k


def _tc_row_major(vocab):
    bn = 8192

    def body(xt_ref, o_ref):
        xt = xt_ref[...]
        rows = lax.broadcasted_iota(jnp.int32, (_EMBED, 128), 0)
        cols = lax.broadcasted_iota(jnp.int32, (_EMBED, 128), 1)
        eye = (rows == cols).astype(jnp.float32)
        o_ref[...] = lax.dot_general(
            xt, eye, (((0,), (0,)), ((), ())),
            preferred_element_type=jnp.float32)

    def apply(tab_t):
        return pl.pallas_call(
            body,
            grid=(pl.cdiv(vocab, bn),),
            in_specs=[pl.BlockSpec((_EMBED, bn), lambda i: (0, i))],
            out_specs=pl.BlockSpec((bn, 128), lambda i: (i, 0)),
            out_shape=jax.ShapeDtypeStruct((vocab, 128), jnp.float32),
        )(tab_t)

    return apply


def _sc_gather(batch, n_workers):
    b_per_w = batch // n_workers
    mesh = plsc.VectorSubcoreMesh(core_axis_name="c", subcore_axis_name="s")

    @functools.partial(
        pl.kernel,
        out_type=jax.ShapeDtypeStruct((6, batch, _EMBED), jnp.float32),
        mesh=mesh,
        compiler_params=pltpu.CompilerParams(use_tc_tiling_on_sc=False),
        scratch_types=[
            pltpu.VMEM((3, b_per_w), jnp.int32),
            pltpu.VMEM((3, _HIST, b_per_w), jnp.int32),
            pltpu.VMEM((3, b_per_w, _EMBED), jnp.float32),
            pltpu.VMEM((3, b_per_w, _EMBED), jnp.float32),
            pltpu.SemaphoreType.DMA,
            pltpu.SemaphoreType.DMA,
        ],
    )
    def k(seed_id, seed_artist_id, seed_album_id,
          hist_item_t, hist_artist_t, hist_album_t,
          item_table, artist_table, album_table,
          out, idx_v, hidx_v, rows_v, acc_v, sem_u, sem_h):
        wid = lax.axis_index("s") * 2 + lax.axis_index("c")
        base = wid * b_per_w

        pltpu.sync_copy(seed_id.at[pl.ds(base, b_per_w)], idx_v.at[0])
        pltpu.sync_copy(seed_artist_id.at[pl.ds(base, b_per_w)], idx_v.at[1])
        pltpu.sync_copy(seed_album_id.at[pl.ds(base, b_per_w)], idx_v.at[2])
        pltpu.sync_copy(hist_item_t.at[:, pl.ds(base, b_per_w)], hidx_v.at[0])
        pltpu.sync_copy(hist_artist_t.at[:, pl.ds(base, b_per_w)], hidx_v.at[1])
        pltpu.sync_copy(hist_album_t.at[:, pl.ds(base, b_per_w)], hidx_v.at[2])

        zero = jnp.zeros((_LANES,), jnp.float32)

        def zero_body(i, _):
            for t in range(3):
                for c in range(_EMBED // _LANES):
                    acc_v[t, i, pl.ds(c * _LANES, _LANES)] = zero
            return 0

        lax.fori_loop(0, b_per_w, zero_body, 0)

        tables = (item_table, artist_table, album_table)

        ds = [pltpu.async_copy(tables[t].at[idx_v.at[t]], rows_v.at[t], sem_u)
              for t in range(3)]

        def issue_body(j, _):
            for t in range(3):
                pltpu.async_copy(
                    tables[t].at[hidx_v.at[t, j]], acc_v.at[t], sem_h, add=True)
            return 0

        lax.fori_loop(0, _HIST, issue_body, 0)

        for d in ds:
            d.wait()
        for t in range(3):
            pltpu.sync_copy(rows_v.at[t], out.at[3 + t, pl.ds(base, b_per_w)])

        def drain_body(j, _):
            for t in range(3):
                pltpu.make_async_copy(
                    tables[t].at[hidx_v.at[t, j]], acc_v.at[t], sem_h).wait()
            return 0

        lax.fori_loop(0, _HIST, drain_body, 0)

        scale = jnp.full((_LANES,), 1.0 / _HIST, jnp.float32)

        def scale_body(i, _):
            for t in range(3):
                for c in range(_EMBED // _LANES):
                    sl = pl.ds(c * _LANES, _LANES)
                    acc_v[t, i, sl] = acc_v[t, i, sl] * scale
            return 0

        lax.fori_loop(0, b_per_w, scale_body, 0)
        for t in range(3):
            pltpu.sync_copy(acc_v.at[t], out.at[t, pl.ds(base, b_per_w)])

    return k


def _mlp_body(u_ref, x_ref, htl_ref, sl_ref, age_ref, w1_ref, ws_ref, b1_ref,
              w2_ref, b2_ref, out_ref):
    xcat = jnp.concatenate(
        [u_ref[...]] + [x_ref[t] for t in range(6)], axis=1)
    s = jnp.dot(xcat, w1_ref[...], preferred_element_type=jnp.float32,
                precision=lax.Precision.HIGHEST)
    hl = jnp.sum(htl_ref[...], axis=1, keepdims=True) * (1.0 / _HIST)
    age_n = jnp.log1p(age_ref[...]) - _LOG_AGE_MEAN
    s = s + hl * ws_ref[0:1, :] + sl_ref[...] * ws_ref[1:2, :] \
        + age_n * ws_ref[2:3, :]
    h = jnp.maximum(s + b1_ref[...], 0.0)
    out_ref[...] = jnp.dot(h, w2_ref[...], preferred_element_type=jnp.float32,
                           precision=lax.Precision.HIGHEST) + b2_ref[...]


def kernel(user_id, history_id, history_artist_id, history_album_id,
           history_track_length, seed_id, seed_artist_id, seed_album_id,
           seed_track_length, age,
           user_table, item_table, artist_table, album_table,
           W1, b1, W2, b2):
    batch = user_id.shape[0]
    n_workers = 32
    def _row_major(tab):
        vocab = tab.shape[0]
        rm = _tc_row_major(vocab)(tab.T)
        return jnp.reshape(rm, (2 * vocab, _EMBED))

    item_rm = _row_major(item_table)
    artist_rm = _row_major(artist_table)
    album_rm = _row_major(album_table)
    emb = _sc_gather(batch, n_workers)(
        seed_id * 2, seed_artist_id * 2, seed_album_id * 2,
        history_id.T * 2, history_artist_id.T * 2, history_album_id.T * 2,
        item_rm, artist_rm, album_rm)

    user_rows = _sc_user_gather(batch, n_workers, user_table.shape[0])(
        user_table.T, user_id)

    w1_emb = jnp.concatenate([W1[0:256], W1[257:449]], axis=0)
    w_scal = jnp.stack([W1[256], W1[449], W1[450]], axis=0)

    hidden1 = W1.shape[1]
    hidden2 = W2.shape[1]
    bm = 512
    grid = (batch // bm,)
    out = pl.pallas_call(
        _mlp_body,
        grid=grid,
        in_specs=[
            pl.BlockSpec((bm, _EMBED), lambda i: (i, 0)),
            pl.BlockSpec((6, bm, _EMBED), lambda i: (0, i, 0)),
            pl.BlockSpec((bm, _HIST), lambda i: (i, 0)),
            pl.BlockSpec((bm, 1), lambda i: (i, 0)),
            pl.BlockSpec((bm, 1), lambda i: (i, 0)),
            pl.BlockSpec((7 * _EMBED, hidden1), lambda i: (0, 0)),
            pl.BlockSpec((3, hidden1), lambda i: (0, 0)),
            pl.BlockSpec((1, hidden1), lambda i: (0, 0)),
            pl.BlockSpec((hidden1, hidden2), lambda i: (0, 0)),
            pl.BlockSpec((1, hidden2), lambda i: (0, 0)),
        ],
        out_specs=pl.BlockSpec((bm, hidden2), lambda i: (i, 0)),
        out_shape=jax.ShapeDtypeStruct((batch, hidden2), jnp.float32),
    )(user_rows, emb, history_track_length, seed_track_length[:, None],
      age[:, None], w1_emb, w_scal, b1[None, :], W2, b2[None, :])
    return out


# speedup vs baseline: 3.8167x; 1.0064x over previous
import functools

import jax
import jax.numpy as jnp
from jax import lax
from jax.experimental import pallas as pl
from jax.experimental.pallas import tpu as pltpu
from jax.experimental.pallas import tpu_sc as plsc

_EMBED = 64
_HIST = 50
_LANES = 16
_LOG_AGE_MEAN = 3.0


def _sc_user_gather(batch, n_workers, vocab):
    b_per_w = batch // n_workers
    k_inflight = 4
    mesh = plsc.VectorSubcoreMesh(core_axis_name="c", subcore_axis_name="s")

    @functools.partial(
        pl.kernel,
        out_type=jax.ShapeDtypeStruct((batch, _EMBED), jnp.float32),
        mesh=mesh,
        compiler_params=pltpu.CompilerParams(needs_layout_passes=False),
        scratch_types=[
            pltpu.VMEM((b_per_w + _LANES,), jnp.int32),
            pltpu.VMEM((2, k_inflight, _EMBED, 128), jnp.float32),
            pltpu.VMEM((b_per_w, _EMBED), jnp.float32),
            pltpu.SemaphoreType.DMA,
            pltpu.SemaphoreType.DMA,
        ],
    )
    def k(user_table_t, user_id, out, idx_v, tiles_v, rows_v, sem, sem2):
        wid = lax.axis_index("s") * 2 + lax.axis_index("c")
        base = wid * b_per_w
        pltpu.sync_copy(user_id.at[pl.ds(base, b_per_w)],
                        idx_v.at[pl.ds(0, b_per_w)])
        eight = _EMBED // _LANES

        n_rounds = b_per_w // k_inflight
        iota = lax.iota(jnp.int32, _LANES)

        def fire(r, buf):
            vec = idx_v[pl.ds(r * k_inflight, _LANES)]
            for l in range(k_inflight):
                tile = pl.multiple_of((vec[l] >> 7) << 7, 128)
                pltpu.async_copy(
                    user_table_t.at[:, pl.ds(tile, 128)],
                    tiles_v.at[buf, l], sem)

        def extract(r, buf):
            vec = idx_v[pl.ds(r * k_inflight, _LANES)]
            for l in range(k_inflight):
                pltpu.make_async_copy(
                    user_table_t.at[:, pl.ds(0, 128)],
                    tiles_v.at[buf, l], sem).wait()
            for l in range(k_inflight):
                col = jnp.full((_LANES,), vec[l] & 127, jnp.int32)
                for c in range(eight):
                    e_vec = c * _LANES + iota
                    vals = plsc.load_gather(tiles_v.at[buf, l], [e_vec, col])
                    rows_v[r * k_inflight + l, pl.ds(c * _LANES, _LANES)] = vals

        fire(0, 0)

        def body(r, _):
            buf = r & 1
            @pl.when(r + 1 < n_rounds)
            def _():
                fire(r + 1, (r + 1) & 1)
            extract(r, buf)
            return 0

        lax.fori_loop(0, n_rounds, body, 0, unroll=2)
        pltpu.sync_copy(rows_v, out.at[pl.ds(base, b_per_w)])

    return k


def _tc_row_major(vocab):
    bn = 8192

    def body(xt_ref, o_ref):
        xt = xt_ref[...]
        rows = lax.broadcasted_iota(jnp.int32, (_EMBED, 128), 0)
        cols = lax.broadcasted_iota(jnp.int32, (_EMBED, 128), 1)
        eye = (rows == cols).astype(jnp.float32)
        o_ref[...] = lax.dot_general(
            xt, eye, (((0,), (0,)), ((), ())),
            preferred_element_type=jnp.float32)

    def apply(tab_t):
        return pl.pallas_call(
            body,
            grid=(pl.cdiv(vocab, bn),),
            in_specs=[pl.BlockSpec((_EMBED, bn), lambda i: (0, i))],
            out_specs=pl.BlockSpec((bn, 128), lambda i: (i, 0)),
            out_shape=jax.ShapeDtypeStruct((vocab, 128), jnp.float32),
        )(tab_t)

    return apply


def _sc_gather(batch, n_workers):
    b_per_w = batch // n_workers
    mesh = plsc.VectorSubcoreMesh(core_axis_name="c", subcore_axis_name="s")

    @functools.partial(
        pl.kernel,
        out_type=jax.ShapeDtypeStruct((6, batch, _EMBED), jnp.float32),
        mesh=mesh,
        compiler_params=pltpu.CompilerParams(use_tc_tiling_on_sc=False),
        scratch_types=[
            pltpu.VMEM((3, b_per_w), jnp.int32),
            pltpu.VMEM((3, _HIST, b_per_w), jnp.int32),
            pltpu.VMEM((3, b_per_w, _EMBED), jnp.float32),
            pltpu.VMEM((3, b_per_w, _EMBED), jnp.float32),
            pltpu.SemaphoreType.DMA,
            pltpu.SemaphoreType.DMA,
        ],
    )
    def k(seed_id, seed_artist_id, seed_album_id,
          hist_item_t, hist_artist_t, hist_album_t,
          item_table, artist_table, album_table, order_dep,
          out, idx_v, hidx_v, rows_v, acc_v, sem_u, sem_h):
        del order_dep
        wid = lax.axis_index("s") * 2 + lax.axis_index("c")
        base = wid * b_per_w

        pltpu.sync_copy(seed_id.at[pl.ds(base, b_per_w)], idx_v.at[0])
        pltpu.sync_copy(seed_artist_id.at[pl.ds(base, b_per_w)], idx_v.at[1])
        pltpu.sync_copy(seed_album_id.at[pl.ds(base, b_per_w)], idx_v.at[2])
        pltpu.sync_copy(hist_item_t.at[:, pl.ds(base, b_per_w)], hidx_v.at[0])
        pltpu.sync_copy(hist_artist_t.at[:, pl.ds(base, b_per_w)], hidx_v.at[1])
        pltpu.sync_copy(hist_album_t.at[:, pl.ds(base, b_per_w)], hidx_v.at[2])

        zero = jnp.zeros((_LANES,), jnp.float32)

        def zero_body(i, _):
            for t in range(3):
                for c in range(_EMBED // _LANES):
                    acc_v[t, i, pl.ds(c * _LANES, _LANES)] = zero
            return 0

        lax.fori_loop(0, b_per_w, zero_body, 0)

        tables = (item_table, artist_table, album_table)

        ds = [pltpu.async_copy(tables[t].at[idx_v.at[t]], rows_v.at[t], sem_u)
              for t in range(3)]

        def issue_body(j, _):
            for t in range(3):
                pltpu.async_copy(
                    tables[t].at[hidx_v.at[t, j]], acc_v.at[t], sem_h, add=True)
            return 0

        lax.fori_loop(0, _HIST, issue_body, 0)

        for d in ds:
            d.wait()
        for t in range(3):
            pltpu.sync_copy(rows_v.at[t], out.at[3 + t, pl.ds(base, b_per_w)])

        def drain_body(j, _):
            for t in range(3):
                pltpu.make_async_copy(
                    tables[t].at[hidx_v.at[t, j]], acc_v.at[t], sem_h).wait()
            return 0

        lax.fori_loop(0, _HIST, drain_body, 0)

        scale = jnp.full((_LANES,), 1.0 / _HIST, jnp.float32)

        def scale_body(i, _):
            for t in range(3):
                for c in range(_EMBED // _LANES):
                    sl = pl.ds(c * _LANES, _LANES)
                    acc_v[t, i, sl] = acc_v[t, i, sl] * scale
            return 0

        lax.fori_loop(0, b_per_w, scale_body, 0)
        for t in range(3):
            pltpu.sync_copy(acc_v.at[t], out.at[t, pl.ds(base, b_per_w)])

    return k


def _mlp_body(u_ref, x_ref, htl_ref, sl_ref, age_ref, w1_ref, ws_ref, b1_ref,
              w2_ref, b2_ref, out_ref):
    xcat = jnp.concatenate(
        [u_ref[...]] + [x_ref[t] for t in range(6)], axis=1)
    s = jnp.dot(xcat, w1_ref[...], preferred_element_type=jnp.float32,
                precision=lax.Precision.HIGHEST)
    hl = jnp.sum(htl_ref[...], axis=1, keepdims=True) * (1.0 / _HIST)
    age_n = jnp.log1p(age_ref[...]) - _LOG_AGE_MEAN
    s = s + hl * ws_ref[0:1, :] + sl_ref[...] * ws_ref[1:2, :] \
        + age_n * ws_ref[2:3, :]
    h = jnp.maximum(s + b1_ref[...], 0.0)
    out_ref[...] = jnp.dot(h, w2_ref[...], preferred_element_type=jnp.float32,
                           precision=lax.Precision.HIGHEST) + b2_ref[...]


def kernel(user_id, history_id, history_artist_id, history_album_id,
           history_track_length, seed_id, seed_artist_id, seed_album_id,
           seed_track_length, age,
           user_table, item_table, artist_table, album_table,
           W1, b1, W2, b2):
    batch = user_id.shape[0]
    n_workers = 32
    def _row_major(tab):
        vocab = tab.shape[0]
        rm = _tc_row_major(vocab)(tab.T)
        return jnp.reshape(rm, (2 * vocab, _EMBED))

    user_rows = _sc_user_gather(batch, n_workers, user_table.shape[0])(
        user_table.T, user_id)
    item_rm = _row_major(item_table)
    artist_rm = _row_major(artist_table)
    album_rm = _row_major(album_table)
    emb = _sc_gather(batch, n_workers)(
        seed_id * 2, seed_artist_id * 2, seed_album_id * 2,
        history_id.T * 2, history_artist_id.T * 2, history_album_id.T * 2,
        item_rm, artist_rm, album_rm, user_rows)

    w1_emb = jnp.concatenate([W1[0:256], W1[257:449]], axis=0)
    w_scal = jnp.stack([W1[256], W1[449], W1[450]], axis=0)

    hidden1 = W1.shape[1]
    hidden2 = W2.shape[1]
    bm = 512
    grid = (batch // bm,)
    out = pl.pallas_call(
        _mlp_body,
        grid=grid,
        in_specs=[
            pl.BlockSpec((bm, _EMBED), lambda i: (i, 0)),
            pl.BlockSpec((6, bm, _EMBED), lambda i: (0, i, 0)),
            pl.BlockSpec((bm, _HIST), lambda i: (i, 0)),
            pl.BlockSpec((bm, 1), lambda i: (i, 0)),
            pl.BlockSpec((bm, 1), lambda i: (i, 0)),
            pl.BlockSpec((7 * _EMBED, hidden1), lambda i: (0, 0)),
            pl.BlockSpec((3, hidden1), lambda i: (0, 0)),
            pl.BlockSpec((1, hidden1), lambda i: (0, 0)),
            pl.BlockSpec((hidden1, hidden2), lambda i: (0, 0)),
            pl.BlockSpec((1, hidden2), lambda i: (0, 0)),
        ],
        out_specs=pl.BlockSpec((bm, hidden2), lambda i: (i, 0)),
        out_shape=jax.ShapeDtypeStruct((batch, hidden2), jnp.float32),
    )(user_rows, emb, history_track_length, seed_track_length[:, None],
      age[:, None], w1_emb, w_scal, b1[None, :], W2, b2[None, :])
    return out
